# baseline (speedup 1.0000x reference)
import functools

import jax
import jax.numpy as jnp
from jax import lax
from jax.experimental import pallas as pl
from jax.experimental.pallas import tpu as pltpu, tpu_sc as plsc

LEN_RULES = 100000
PAD_TOKEN = 100000
B = 16384
L = 200
NUM_CORES = 2
NUM_SUBCORES = 16
NW = NUM_CORES * NUM_SUBCORES
ROWS_PER_W = B // NW
CHUNK_ROWS = 128
NCHUNK = ROWS_PER_W // CHUNK_ROWS
GROUPS = CHUNK_ROWS // 16
PARTS = ((0, 72), (72, 64), (136, 64))
T_PAD = 100096
T_SLICE = T_PAD // NUM_SUBCORES


def _sc_body(rules_hbm, table_hbm, out_hbm,
             table_f, rules_v0, rules_v1, out_v, table_spm, sem0, sem1):
    cid = lax.axis_index("c")
    sid = lax.axis_index("s")
    wid = sid * NUM_CORES + cid
    base_row = wid * ROWS_PER_W

    bufs = (rules_v0, rules_v1)
    sems = (sem0, sem1)
    steps = [(c, off, ln) for c in range(NCHUNK) for off, ln in PARTS]

    def start(i):
        c, off, ln = steps[i]
        return pltpu.async_copy(
            rules_hbm.at[pl.ds(off, ln),
                         pl.ds(base_row + c * CHUNK_ROWS, CHUNK_ROWS)],
            bufs[i % 2].at[pl.ds(0, ln), :],
            sems[i % 2],
        )

    pending = start(0)

    t_off = sid * T_SLICE
    pltpu.sync_copy(table_hbm.at[pl.ds(t_off, T_SLICE)],
                    table_f.at[pl.ds(t_off, T_SLICE)])

    def xform(i, _):
        o = t_off + i * 16
        table_f[pl.ds(o, 16)] = 1.0 + jnp.exp(table_f[pl.ds(o, 16)])
        return 0

    lax.fori_loop(0, T_SLICE // 16, xform, 0, unroll=8)

    @pl.when(sid == PAD_TOKEN // T_SLICE)
    def _patch():
        v = table_f[pl.ds(PAD_TOKEN, 16)]
        v = jnp.where(lax.iota(jnp.int32, 16) == PAD_TOKEN % 16, 1.0, v)
        table_f[pl.ds(PAD_TOKEN, 16)] = v

    pltpu.sync_copy(table_f.at[pl.ds(t_off, T_SLICE)],
                    table_spm.at[pl.ds(t_off, T_SLICE)])
    plsc.subcore_barrier()
    pltpu.sync_copy(table_spm, table_f)

    dens = None
    for i, (c, off, ln) in enumerate(steps):
        pending.wait()
        if i + 1 < len(steps):
            nxt = start(i + 1)
        rules_v = bufs[i % 2]
        if off == 0:
            dens = [jnp.ones((16,), jnp.float32)] * GROUPS
        for gp in range(GROUPS // 4):
            col0 = gp * 64

            def step(l, dd):
                new = []
                for q in range(4):
                    rv = rules_v[l, pl.ds(col0 + q * 16, 16)]
                    f = plsc.load_gather(table_f, [rv])
                    new.append(dd[q] * f)
                return tuple(new)

            (dens[4 * gp], dens[4 * gp + 1],
             dens[4 * gp + 2], dens[4 * gp + 3]) = lax.fori_loop(
                0, ln, step,
                (dens[4 * gp], dens[4 * gp + 1],
                 dens[4 * gp + 2], dens[4 * gp + 3]), unroll=4)
        if off + ln == L:
            for g in range(GROUPS):
                res = jnp.clip(1.0 - 1.0 / dens[g], 0.0001, 0.99999)
                out_v[pl.ds(c * CHUNK_ROWS + g * 16, 16)] = res
        if i + 1 < len(steps):
            pending = nxt

    pltpu.sync_copy(out_v, out_hbm.at[pl.ds(base_row, ROWS_PER_W)])


@functools.partial(jax.jit, static_argnames=())
def kernel(rules, relation, table):
    del relation
    table_p = jnp.pad(table.reshape(-1), (0, T_PAD - (LEN_RULES + 1)))
    mesh = plsc.VectorSubcoreMesh(core_axis_name="c", subcore_axis_name="s")
    out = pl.kernel(
        _sc_body,
        out_type=jax.ShapeDtypeStruct((B,), jnp.float32),
        mesh=mesh,
        scratch_types=[
            pltpu.VMEM((T_PAD,), jnp.float32),
            pltpu.VMEM((72, CHUNK_ROWS), jnp.int32),
            pltpu.VMEM((72, CHUNK_ROWS), jnp.int32),
            pltpu.VMEM((ROWS_PER_W,), jnp.float32),
            pltpu.VMEM_SHARED((T_PAD,), jnp.float32),
            pltpu.SemaphoreType.DMA,
            pltpu.SemaphoreType.DMA,
        ],
        compiler_params=pltpu.CompilerParams(needs_layout_passes=False),
    )(rules.T, table_p)
    return out.reshape(B, 1)


# speedup vs baseline: 1.0403x; 1.0403x over previous
import functools

import jax
import jax.numpy as jnp
from jax import lax
from jax.experimental import pallas as pl
from jax.experimental.pallas import tpu as pltpu, tpu_sc as plsc

LEN_RULES = 100000
PAD_TOKEN = 100000
B = 16384
L = 200
NUM_CORES = 2
NUM_SUBCORES = 16
NW = NUM_CORES * NUM_SUBCORES
ROWS_PER_W = B // NW
CHUNK_ROWS = 128
NCHUNK = ROWS_PER_W // CHUNK_ROWS
GROUPS = CHUNK_ROWS // 16
PARTS = ((0, 72), (72, 64), (136, 64))
T_PAD = 100096
T_SLICE = T_PAD // NUM_SUBCORES


def _sc_body(rules_hbm, table_hbm, out_hbm,
             table_f, rules_v0, rules_v1, out_v, table_spm, sem0, sem1):
    cid = lax.axis_index("c")
    sid = lax.axis_index("s")
    wid = sid * NUM_CORES + cid
    base_row = wid * ROWS_PER_W

    bufs = (rules_v0, rules_v1)
    sems = (sem0, sem1)
    steps = [(c, off, ln) for c in range(NCHUNK) for off, ln in PARTS]

    def start(i):
        c, off, ln = steps[i]
        return pltpu.async_copy(
            rules_hbm.at[pl.ds(off, ln),
                         pl.ds(base_row + c * CHUNK_ROWS, CHUNK_ROWS)],
            bufs[i % 2].at[pl.ds(0, ln), :],
            sems[i % 2],
        )

    pending = start(0)

    t_off = sid * T_SLICE
    pltpu.sync_copy(table_hbm.at[pl.ds(t_off, T_SLICE)],
                    table_f.at[pl.ds(t_off, T_SLICE)])

    def xform(i, _):
        o = t_off + i * 16
        table_f[pl.ds(o, 16)] = 1.0 + jnp.exp(table_f[pl.ds(o, 16)])
        return 0

    lax.fori_loop(0, T_SLICE // 16, xform, 0, unroll=8)

    @pl.when(sid == PAD_TOKEN // T_SLICE)
    def _patch():
        v = table_f[pl.ds(PAD_TOKEN, 16)]
        v = jnp.where(lax.iota(jnp.int32, 16) == PAD_TOKEN % 16, 1.0, v)
        table_f[pl.ds(PAD_TOKEN, 16)] = v

    pltpu.sync_copy(table_f.at[pl.ds(t_off, T_SLICE)],
                    table_spm.at[pl.ds(t_off, T_SLICE)])
    plsc.subcore_barrier()
    pltpu.sync_copy(table_spm, table_f)

    dens = None
    for i, (c, off, ln) in enumerate(steps):
        pending.wait()
        if i + 1 < len(steps):
            nxt = start(i + 1)
        rules_v = bufs[i % 2]
        if off == 0:
            dens = [jnp.ones((16,), jnp.float32)] * GROUPS
        for gp in range(GROUPS // 4):
            col0 = gp * 64

            def step(l, dd):
                new = []
                for q in range(4):
                    rv = rules_v[l, pl.ds(col0 + q * 16, 16)]
                    f = plsc.load_gather(table_f, [rv])
                    new.append(dd[q] * f)
                return tuple(new)

            (dens[4 * gp], dens[4 * gp + 1],
             dens[4 * gp + 2], dens[4 * gp + 3]) = lax.fori_loop(
                0, ln, step,
                (dens[4 * gp], dens[4 * gp + 1],
                 dens[4 * gp + 2], dens[4 * gp + 3]), unroll=1)
        if off + ln == L:
            for g in range(GROUPS):
                res = jnp.clip(1.0 - 1.0 / dens[g], 0.0001, 0.99999)
                out_v[pl.ds(c * CHUNK_ROWS + g * 16, 16)] = res
        if i + 1 < len(steps):
            pending = nxt

    pltpu.sync_copy(out_v, out_hbm.at[pl.ds(base_row, ROWS_PER_W)])


@functools.partial(jax.jit, static_argnames=())
def kernel(rules, relation, table):
    del relation
    table_p = jnp.pad(table.reshape(-1), (0, T_PAD - (LEN_RULES + 1)))
    mesh = plsc.VectorSubcoreMesh(core_axis_name="c", subcore_axis_name="s")
    out = pl.kernel(
        _sc_body,
        out_type=jax.ShapeDtypeStruct((B,), jnp.float32),
        mesh=mesh,
        scratch_types=[
            pltpu.VMEM((T_PAD,), jnp.float32),
            pltpu.VMEM((72, CHUNK_ROWS), jnp.int32),
            pltpu.VMEM((72, CHUNK_ROWS), jnp.int32),
            pltpu.VMEM((ROWS_PER_W,), jnp.float32),
            pltpu.VMEM_SHARED((T_PAD,), jnp.float32),
            pltpu.SemaphoreType.DMA,
            pltpu.SemaphoreType.DMA,
        ],
        compiler_params=pltpu.CompilerParams(needs_layout_passes=False),
    )(rules.T, table_p)
    return out.reshape(B, 1)


# speedup vs baseline: 1.0433x; 1.0029x over previous
import functools

import jax
import jax.numpy as jnp
from jax import lax
from jax.experimental import pallas as pl
from jax.experimental.pallas import tpu as pltpu, tpu_sc as plsc

LEN_RULES = 100000
PAD_TOKEN = 100000
B = 16384
L = 200
NUM_CORES = 2
NUM_SUBCORES = 16
NW = NUM_CORES * NUM_SUBCORES
ROWS_PER_W = B // NW
CHUNK_ROWS = 128
NCHUNK = ROWS_PER_W // CHUNK_ROWS
GROUPS = CHUNK_ROWS // 16
PARTS = ((0, 72), (72, 64), (136, 64))
T_PAD = 100096
T_SLICE = T_PAD // NUM_SUBCORES


def _sc_body(rules_hbm, table_hbm, out_hbm,
             table_f, rules_v0, rules_v1, out_v, table_spm, sem0, sem1):
    cid = lax.axis_index("c")
    sid = lax.axis_index("s")
    wid = sid * NUM_CORES + cid
    base_row = wid * ROWS_PER_W

    bufs = (rules_v0, rules_v1)
    sems = (sem0, sem1)
    steps = [(c, off, ln) for c in range(NCHUNK) for off, ln in PARTS]

    def start(i):
        c, off, ln = steps[i]
        return pltpu.async_copy(
            rules_hbm.at[pl.ds(off, ln),
                         pl.ds(base_row + c * CHUNK_ROWS, CHUNK_ROWS)],
            bufs[i % 2].at[pl.ds(0, ln), :],
            sems[i % 2],
        )

    pending = start(0)

    t_off = sid * T_SLICE
    pltpu.sync_copy(table_hbm.at[pl.ds(t_off, T_SLICE)],
                    table_f.at[pl.ds(t_off, T_SLICE)])

    def xform(i, _):
        o = t_off + i * 16
        table_f[pl.ds(o, 16)] = 1.0 + jnp.exp(table_f[pl.ds(o, 16)])
        return 0

    lax.fori_loop(0, T_SLICE // 16, xform, 0, unroll=8)

    @pl.when(sid == PAD_TOKEN // T_SLICE)
    def _patch():
        v = table_f[pl.ds(PAD_TOKEN, 16)]
        v = jnp.where(lax.iota(jnp.int32, 16) == PAD_TOKEN % 16, 1.0, v)
        table_f[pl.ds(PAD_TOKEN, 16)] = v

    pltpu.sync_copy(table_f.at[pl.ds(t_off, T_SLICE)],
                    table_spm.at[pl.ds(t_off, T_SLICE)])
    plsc.subcore_barrier()
    pltpu.sync_copy(table_spm, table_f)

    dens = None
    for i, (c, off, ln) in enumerate(steps):
        pending.wait()
        if i + 1 < len(steps):
            nxt = start(i + 1)
        rules_v = bufs[i % 2]
        if off == 0:
            dens = [jnp.ones((16,), jnp.float32)] * GROUPS
        for gp in range(GROUPS // 4):
            col0 = gp * 64

            def step(l, dd):
                new = []
                for q in range(4):
                    rv = rules_v[l, pl.ds(col0 + q * 16, 16)]
                    f = plsc.load_gather(table_f, [rv])
                    new.append(dd[q] * f)
                return tuple(new)

            (dens[4 * gp], dens[4 * gp + 1],
             dens[4 * gp + 2], dens[4 * gp + 3]) = lax.fori_loop(
                0, ln, step,
                (dens[4 * gp], dens[4 * gp + 1],
                 dens[4 * gp + 2], dens[4 * gp + 3]), unroll=2)
        if off + ln == L:
            for g in range(GROUPS):
                res = jnp.clip(1.0 - 1.0 / dens[g], 0.0001, 0.99999)
                out_v[pl.ds(c * CHUNK_ROWS + g * 16, 16)] = res
        if i + 1 < len(steps):
            pending = nxt

    pltpu.sync_copy(out_v, out_hbm.at[pl.ds(base_row, ROWS_PER_W)])


@functools.partial(jax.jit, static_argnames=())
def kernel(rules, relation, table):
    del relation
    table_p = jnp.pad(table.reshape(-1), (0, T_PAD - (LEN_RULES + 1)))
    mesh = plsc.VectorSubcoreMesh(core_axis_name="c", subcore_axis_name="s")
    out = pl.kernel(
        _sc_body,
        out_type=jax.ShapeDtypeStruct((B,), jnp.float32),
        mesh=mesh,
        scratch_types=[
            pltpu.VMEM((T_PAD,), jnp.float32),
            pltpu.VMEM((72, CHUNK_ROWS), jnp.int32),
            pltpu.VMEM((72, CHUNK_ROWS), jnp.int32),
            pltpu.VMEM((ROWS_PER_W,), jnp.float32),
            pltpu.VMEM_SHARED((T_PAD,), jnp.float32),
            pltpu.SemaphoreType.DMA,
            pltpu.SemaphoreType.DMA,
        ],
        compiler_params=pltpu.CompilerParams(needs_layout_passes=False,
                                             skip_device_barrier=True),
    )(rules.T, table_p)
    return out.reshape(B, 1)
